# baseline (speedup 1.0000x reference)
import functools

import jax
import jax.numpy as jnp
import numpy as np
from jax import lax
from jax.experimental import pallas as pl
from jax.experimental.pallas import tpu as pltpu
from jax.experimental.pallas import tpu_sc as plsc

KK = 9
NN = 64
SC = 2
LL = 112 * 112
TL = 896
NBLK = LL // TL

_SIDX = tuple(
    int(v) for v in np.round(
        np.arange(NN, dtype=np.float32)
        * (np.float32(LL - 1) / np.float32(NN - 1))).astype(np.int32))


def _unshuffle(x, s):
    B, C, H, W = x.shape
    x = x.reshape(B, C, H // s, s, W // s, s)
    x = x.transpose(0, 1, 3, 5, 2, 4)
    return x.reshape(B, C * s * s, H // s, W // s)


def _shuffle(x, s):
    B, C, H, W = x.shape
    x = x.reshape(B, C // (s * s), s, s, H, W)
    x = x.transpose(0, 1, 4, 2, 5, 3)
    return x.reshape(B, C // (s * s), H * s, W * s)


def _select_kernel(tok_ref, samp_ref, idx_ref):
    bb = pl.program_id(0)
    tok = tok_ref[0]
    samp = samp_ref[0]
    cross = jax.lax.dot_general(
        tok, samp, (((1,), (1,)), ((), ())),
        preferred_element_type=jnp.float32)
    t2 = jnp.sum(tok * tok, axis=1, keepdims=True)
    s2 = jnp.sum(samp * samp, axis=1)
    d2 = (t2 - 2.0 * cross) + s2[None, :]
    iota = jax.lax.broadcasted_iota(jnp.int32, (TL, NN), 1)
    for k in range(KK):
        m = jnp.min(d2, axis=1, keepdims=True)
        ismin = d2 <= m
        first = jnp.min(jnp.where(ismin, iota, NN), axis=1, keepdims=True)
        d2 = jnp.where(iota == first, jnp.float32(jnp.inf), d2)
        idx_ref[0, k, :, :] = first + bb * NN
    del ismin


def _conv_kernel(relu, nb_ref, wt_ref, bias_ref, out_ref):
    acc = jnp.zeros((TL, wt_ref.shape[2]), jnp.float32)
    for k in range(KK):
        acc = acc + jax.lax.dot_general(
            nb_ref[0, k], wt_ref[k], (((1,), (0,)), ((), ())),
            preferred_element_type=jnp.float32)
    acc = acc + bias_ref[0][None, :]
    if relu:
        acc = jnp.maximum(acc, 0.0)
    out_ref[0] = acc


_CH = 112


def _make_gather(B, Cinp):
    npw = B * LL // 32
    nch = npw // _CH
    mesh = plsc.VectorSubcoreMesh(core_axis_name="c", subcore_axis_name="s")

    @functools.partial(
        pl.kernel, mesh=mesh,
        out_type=jax.ShapeDtypeStruct((B, KK, LL, Cinp), jnp.float32),
        compiler_params=pltpu.CompilerParams(use_tc_tiling_on_sc=False),
        scratch_types=[
            pltpu.VMEM((_CH,), jnp.int32),
            pltpu.VMEM((_CH, Cinp), jnp.float32),
            pltpu.SemaphoreType.DMA,
        ],
    )
    def gather(samp_hbm, idx_hbm, nb_hbm, idx_v, rows_v, sem):
        wid = lax.axis_index("s") * 2 + lax.axis_index("c")
        wpb = 16
        bb = wid // wpb
        base = (wid % wpb) * npw
        for c in range(nch):
            off = base + c * _CH
            for k in range(KK):
                pltpu.sync_copy(
                    idx_hbm.at[pl.ds(k * B * LL + bb * LL + off, _CH)], idx_v)
                pltpu.async_copy(samp_hbm.at[idx_v], rows_v, sem).wait()
                pltpu.sync_copy(rows_v, nb_hbm.at[bb, k, pl.ds(off, _CH)])

    return gather


def _nn_layer(tokens, Wk, b, relu):
    B, L, Cin = tokens.shape
    Cout = Wk.shape[0]
    Cinp = ((Cin + 15) // 16) * 16
    samp = tokens[:, _SIDX, :]
    sampflat = jnp.pad(samp.reshape(B * NN, Cin), ((0, 0), (0, Cinp - Cin)))
    idx = pl.pallas_call(
        _select_kernel,
        grid=(B, NBLK),
        in_specs=[
            pl.BlockSpec((1, TL, Cin), lambda b_, i: (b_, i, 0)),
            pl.BlockSpec((1, NN, Cin), lambda b_, i: (b_, 0, 0)),
        ],
        out_specs=pl.BlockSpec((1, KK, TL, 1), lambda b_, i: (b_, 0, i, 0)),
        out_shape=jax.ShapeDtypeStruct((B, KK, L, 1), jnp.int32),
    )(tokens, samp)
    idx = idx.reshape(B, KK, L).transpose(1, 0, 2).reshape(KK * B * L)
    nb = _make_gather(B, Cinp)(sampflat, idx)
    Wt = jnp.transpose(Wk, (2, 1, 0))
    Wtp = jnp.pad(Wt, ((0, 0), (0, Cinp - Cin), (0, 0)))
    out = pl.pallas_call(
        functools.partial(_conv_kernel, relu),
        grid=(B, NBLK),
        in_specs=[
            pl.BlockSpec((1, KK, TL, Cinp), lambda b_, i: (b_, 0, i, 0)),
            pl.BlockSpec((KK, Cinp, Cout), lambda b_, i: (0, 0, 0)),
            pl.BlockSpec((1, Cout), lambda b_, i: (0, 0)),
        ],
        out_specs=pl.BlockSpec((1, TL, Cout), lambda b_, i: (b_, i, 0)),
        out_shape=jax.ShapeDtypeStruct((B, L, Cout), jnp.float32),
    )(nb, Wtp, b.reshape(1, Cout))
    return out


def kernel(x, W1, b1, W2, b2, W3, b3):
    B = x.shape[0]
    H = x.shape[2]
    W = x.shape[3]
    ys = jnp.linspace(-1.0, 1.0, H)
    xs = jnp.linspace(-1.0, 1.0, W)
    gy, gx = jnp.meshgrid(ys, xs, indexing='ij')
    loc = jnp.stack([gy, gx])[None]
    locu = _unshuffle(loc, SC)
    L = (H // SC) * (W // SC)
    loct = jnp.broadcast_to(
        locu.reshape(1, 2 * SC * SC, L).transpose(0, 2, 1), (B, L, 2 * SC * SC))

    xt = _unshuffle(x, SC)
    t = jnp.concatenate(
        [xt.reshape(B, -1, L).transpose(0, 2, 1), loct], axis=-1)

    o1 = _nn_layer(t, W1, b1, relu=True)
    t = jnp.concatenate([o1, loct], axis=-1)
    o2 = _nn_layer(t, W2, b2, relu=True)
    t = jnp.concatenate([o2, loct], axis=-1)
    o3 = _nn_layer(t, W3, b3, relu=False)

    Cout = o3.shape[-1]
    out = o3.transpose(0, 2, 1).reshape(B, Cout, H // SC, W // SC)
    return _shuffle(out, SC)


# speedup vs baseline: 1.7808x; 1.7808x over previous
import functools

import jax
import jax.numpy as jnp
import numpy as np
from jax import lax
from jax.experimental import pallas as pl
from jax.experimental.pallas import tpu as pltpu
from jax.experimental.pallas import tpu_sc as plsc

KK = 9
NN = 64
SC = 2
LL = 112 * 112
TL = 896
NBLK = LL // TL

_SIDX = tuple(
    int(v) for v in np.round(
        np.arange(NN, dtype=np.float32)
        * (np.float32(LL - 1) / np.float32(NN - 1))).astype(np.int32))


def _unshuffle(x, s):
    B, C, H, W = x.shape
    x = x.reshape(B, C, H // s, s, W // s, s)
    x = x.transpose(0, 1, 3, 5, 2, 4)
    return x.reshape(B, C * s * s, H // s, W // s)


def _shuffle(x, s):
    B, C, H, W = x.shape
    x = x.reshape(B, C // (s * s), s, s, H, W)
    x = x.transpose(0, 1, 4, 2, 5, 3)
    return x.reshape(B, C // (s * s), H * s, W * s)


def _select_kernel(tok_ref, samp_ref, idx_ref):
    bb = pl.program_id(0)
    tok = tok_ref[0]
    samp = samp_ref[0]
    cross = jax.lax.dot_general(
        tok, samp, (((1,), (1,)), ((), ())),
        preferred_element_type=jnp.float32)
    t2 = jnp.sum(tok * tok, axis=1, keepdims=True)
    s2 = jnp.sum(samp * samp, axis=1)
    d2 = (t2 - 2.0 * cross) + s2[None, :]
    iota = jax.lax.broadcasted_iota(jnp.int32, (TL, NN), 1)
    for k in range(KK):
        m = jnp.min(d2, axis=1, keepdims=True)
        ismin = d2 <= m
        first = jnp.min(jnp.where(ismin, iota, NN), axis=1, keepdims=True)
        d2 = jnp.where(iota == first, jnp.float32(jnp.inf), d2)
        idx_ref[0, k, :, :] = first + (bb * KK + k) * NN


def _tab_kernel(samp_ref, w_ref, tab_ref):
    s = samp_ref[0]
    for k in range(KK):
        tab_ref[0, k] = jax.lax.dot_general(
            s, w_ref[k], (((1,), (0,)), ((), ())),
            preferred_element_type=jnp.float32)


_CH = 112


def _make_gather(B, Coutp, relu):
    npw = B * LL // 32
    nch = npw // _CH
    ncol = Coutp // 16
    mesh = plsc.VectorSubcoreMesh(core_axis_name="c", subcore_axis_name="s")

    @functools.partial(
        pl.kernel, mesh=mesh,
        out_type=jax.ShapeDtypeStruct((B, LL, Coutp), jnp.float32),
        compiler_params=pltpu.CompilerParams(use_tc_tiling_on_sc=False),
        scratch_types=[
            pltpu.VMEM((KK, npw), jnp.int32),
            [pltpu.VMEM((_CH, Coutp), jnp.float32)] * 2,
            [pltpu.VMEM((_CH, Coutp), jnp.float32)] * 2,
            pltpu.VMEM((Coutp,), jnp.float32),
            [pltpu.SemaphoreType.DMA] * 2,
            pltpu.SemaphoreType.DMA,
            [pltpu.SemaphoreType.DMA] * 2,
            pltpu.SemaphoreType.DMA,
        ],
    )
    def gather(tab_hbm, idx_hbm, bias_hbm, out_hbm,
               idx_v, rows, accs, bias_v, gsems, asem, wsems, isem):
        wid = lax.axis_index("s") * 2 + lax.axis_index("c")
        bb = wid // 16
        base = (wid % 16) * npw
        pltpu.sync_copy(bias_hbm, bias_v)
        handles = [
            pltpu.async_copy(
                idx_hbm.at[pl.ds(k * B * LL + bb * LL + base, npw)],
                idx_v.at[k], isem)
            for k in range(KK)]
        for h in handles:
            h.wait()
        wh = [None, None]
        for c in range(nch):
            acc = accs[c % 2]
            if wh[c % 2] is not None:
                wh[c % 2].wait()
            ah = pltpu.async_copy(
                tab_hbm.at[idx_v.at[0, pl.ds(c * _CH, _CH)]], acc, asem)
            gh = pltpu.async_copy(
                tab_hbm.at[idx_v.at[1, pl.ds(c * _CH, _CH)]], rows[1], gsems[1])
            ah.wait()
            for k in range(1, KK):
                if k + 1 < KK:
                    gh_next = pltpu.async_copy(
                        tab_hbm.at[idx_v.at[k + 1, pl.ds(c * _CH, _CH)]],
                        rows[(k + 1) % 2], gsems[(k + 1) % 2])
                else:
                    gh_next = None
                gh.wait()
                gh = gh_next
                src = rows[k % 2]

                def addb(t, carry, acc=acc, src=src):
                    for j in range(ncol):
                        plsc.addupdate(acc.at[t, pl.ds(j * 16, 16)],
                                       src[t, pl.ds(j * 16, 16)])
                    return carry

                lax.fori_loop(0, _CH, addb, 0)

            def fin(t, carry, acc=acc):
                for j in range(ncol):
                    v = acc[t, pl.ds(j * 16, 16)] + bias_v[pl.ds(j * 16, 16)]
                    if relu:
                        v = jnp.maximum(v, 0.0)
                    acc[t, pl.ds(j * 16, 16)] = v
                return carry

            lax.fori_loop(0, _CH, fin, 0)
            wh[c % 2] = pltpu.async_copy(
                acc, out_hbm.at[bb, pl.ds(base + c * _CH, _CH)], wsems[c % 2])
        for h in wh:
            if h is not None:
                h.wait()

    return gather


def _nn_layer(tokens, Wk, b, relu):
    B, L, Cin = tokens.shape
    Cout = Wk.shape[0]
    Coutp = ((Cout + 15) // 16) * 16
    samp = tokens[:, _SIDX, :]
    idx = pl.pallas_call(
        _select_kernel,
        grid=(B, NBLK),
        in_specs=[
            pl.BlockSpec((1, TL, Cin), lambda b_, i: (b_, i, 0)),
            pl.BlockSpec((1, NN, Cin), lambda b_, i: (b_, 0, 0)),
        ],
        out_specs=pl.BlockSpec((1, KK, TL, 1), lambda b_, i: (b_, 0, i, 0)),
        out_shape=jax.ShapeDtypeStruct((B, KK, L, 1), jnp.int32),
    )(tokens, samp)
    idx = idx.reshape(B, KK, L).transpose(1, 0, 2).reshape(KK * B * L)
    Wt = jnp.transpose(Wk, (2, 1, 0))
    Wtp = jnp.pad(Wt, ((0, 0), (0, 0), (0, Coutp - Cout)))
    tab = pl.pallas_call(
        _tab_kernel,
        grid=(B,),
        in_specs=[
            pl.BlockSpec((1, NN, Cin), lambda b_: (b_, 0, 0)),
            pl.BlockSpec((KK, Cin, Coutp), lambda b_: (0, 0, 0)),
        ],
        out_specs=pl.BlockSpec((1, KK, NN, Coutp), lambda b_: (b_, 0, 0, 0)),
        out_shape=jax.ShapeDtypeStruct((B, KK, NN, Coutp), jnp.float32),
    )(samp, Wtp)
    tabflat = tab.reshape(B * KK * NN, Coutp)
    biasp = jnp.pad(b, (0, Coutp - Cout))
    out = _make_gather(B, Coutp, relu)(tabflat, idx, biasp)
    return out[:, :, :Cout] if Coutp != Cout else out


def kernel(x, W1, b1, W2, b2, W3, b3):
    B = x.shape[0]
    H = x.shape[2]
    W = x.shape[3]
    ys = jnp.linspace(-1.0, 1.0, H)
    xs = jnp.linspace(-1.0, 1.0, W)
    gy, gx = jnp.meshgrid(ys, xs, indexing='ij')
    loc = jnp.stack([gy, gx])[None]
    locu = _unshuffle(loc, SC)
    L = (H // SC) * (W // SC)
    loct = jnp.broadcast_to(
        locu.reshape(1, 2 * SC * SC, L).transpose(0, 2, 1), (B, L, 2 * SC * SC))

    xt = _unshuffle(x, SC)
    t = jnp.concatenate(
        [xt.reshape(B, -1, L).transpose(0, 2, 1), loct], axis=-1)

    o1 = _nn_layer(t, W1, b1, relu=True)
    t = jnp.concatenate([o1, loct], axis=-1)
    o2 = _nn_layer(t, W2, b2, relu=True)
    t = jnp.concatenate([o2, loct], axis=-1)
    o3 = _nn_layer(t, W3, b3, relu=False)

    Cout = o3.shape[-1]
    out = o3.transpose(0, 2, 1).reshape(B, Cout, H // SC, W // SC)
    return _shuffle(out, SC)


# speedup vs baseline: 1.9687x; 1.1055x over previous
import functools

import jax
import jax.numpy as jnp
import numpy as np
from jax import lax
from jax.experimental import pallas as pl
from jax.experimental.pallas import tpu as pltpu
from jax.experimental.pallas import tpu_sc as plsc

KK = 9
NN = 64
SC = 2
LL = 112 * 112
TL = 896
NBLK = LL // TL

_SIDX = tuple(
    int(v) for v in np.round(
        np.arange(NN, dtype=np.float32)
        * (np.float32(LL - 1) / np.float32(NN - 1))).astype(np.int32))


def _unshuffle(x, s):
    B, C, H, W = x.shape
    x = x.reshape(B, C, H // s, s, W // s, s)
    x = x.transpose(0, 1, 3, 5, 2, 4)
    return x.reshape(B, C * s * s, H // s, W // s)


def _shuffle(x, s):
    B, C, H, W = x.shape
    x = x.reshape(B, C // (s * s), s, s, H, W)
    x = x.transpose(0, 1, 4, 2, 5, 3)
    return x.reshape(B, C // (s * s), H * s, W * s)


def _select_kernel(tok_ref, loc_ref, sampf_ref, sampl_ref, idx_ref):
    bb = pl.program_id(0)
    tok = tok_ref[0]
    locb = loc_ref[...]
    sampf = sampf_ref[0]
    sampl = sampl_ref[...]
    cross = jax.lax.dot_general(
        tok, sampf, (((1,), (1,)), ((), ())),
        preferred_element_type=jnp.float32) + jax.lax.dot_general(
        locb, sampl, (((1,), (1,)), ((), ())),
        preferred_element_type=jnp.float32)
    t2 = (jnp.sum(tok * tok, axis=1, keepdims=True)
          + jnp.sum(locb * locb, axis=1, keepdims=True))
    s2 = jnp.sum(sampf * sampf, axis=1) + jnp.sum(sampl * sampl, axis=1)
    d2 = (t2 - 2.0 * cross) + s2[None, :]
    iota = jax.lax.broadcasted_iota(jnp.int32, (TL, NN), 1)
    for k in range(KK):
        m = jnp.min(d2, axis=1, keepdims=True)
        ismin = d2 <= m
        first = jnp.min(jnp.where(ismin, iota, NN), axis=1, keepdims=True)
        d2 = jnp.where(iota == first, jnp.float32(jnp.inf), d2)
        idx_ref[k, 0, :, :] = first + (bb * KK + k) * NN


def _tab_kernel(sampf_ref, sampl_ref, wf_ref, wl_ref, tab_ref):
    sf = sampf_ref[0]
    sl = sampl_ref[...]
    for k in range(KK):
        tab_ref[0, k] = jax.lax.dot_general(
            sf, wf_ref[k], (((1,), (0,)), ((), ())),
            preferred_element_type=jnp.float32) + jax.lax.dot_general(
            sl, wl_ref[k], (((1,), (0,)), ((), ())),
            preferred_element_type=jnp.float32)


_CH = 112


def _make_gather(B, Coutp, relu):
    npw = B * LL // 32
    nch = npw // _CH
    ncol = Coutp // 16
    mesh = plsc.VectorSubcoreMesh(core_axis_name="c", subcore_axis_name="s")

    @functools.partial(
        pl.kernel, mesh=mesh,
        out_type=jax.ShapeDtypeStruct((B, LL, Coutp), jnp.float32),
        compiler_params=pltpu.CompilerParams(use_tc_tiling_on_sc=False),
        scratch_types=[
            pltpu.VMEM((KK, npw), jnp.int32),
            [pltpu.VMEM((_CH, Coutp), jnp.float32)] * 2,
            [pltpu.VMEM((_CH, Coutp), jnp.float32)] * 2,
            pltpu.VMEM((Coutp,), jnp.float32),
            [pltpu.SemaphoreType.DMA] * 2,
            pltpu.SemaphoreType.DMA,
            [pltpu.SemaphoreType.DMA] * 2,
            pltpu.SemaphoreType.DMA,
        ],
    )
    def gather(tab_hbm, idx_hbm, bias_hbm, out_hbm,
               idx_v, rows, accs, bias_v, gsems, asem, wsems, isem):
        wid = lax.axis_index("s") * 2 + lax.axis_index("c")
        bb = wid // 16
        base = (wid % 16) * npw
        pltpu.sync_copy(bias_hbm, bias_v)
        handles = [
            pltpu.async_copy(
                idx_hbm.at[pl.ds(k * B * LL + bb * LL + base, npw)],
                idx_v.at[k], isem)
            for k in range(KK)]
        for h in handles:
            h.wait()
        wh = [None, None]
        for c in range(nch):
            acc = accs[c % 2]
            if wh[c % 2] is not None:
                wh[c % 2].wait()
            ah = pltpu.async_copy(
                tab_hbm.at[idx_v.at[0, pl.ds(c * _CH, _CH)]], acc, asem)
            gh = pltpu.async_copy(
                tab_hbm.at[idx_v.at[1, pl.ds(c * _CH, _CH)]], rows[1], gsems[1])
            ah.wait()
            for k in range(1, KK):
                if k + 1 < KK:
                    gh_next = pltpu.async_copy(
                        tab_hbm.at[idx_v.at[k + 1, pl.ds(c * _CH, _CH)]],
                        rows[(k + 1) % 2], gsems[(k + 1) % 2])
                else:
                    gh_next = None
                gh.wait()
                gh = gh_next
                src = rows[k % 2]

                def addb(t, carry, acc=acc, src=src):
                    for j in range(ncol):
                        plsc.addupdate(acc.at[t, pl.ds(j * 16, 16)],
                                       src[t, pl.ds(j * 16, 16)])
                    return carry

                lax.fori_loop(0, _CH, addb, 0)

            def fin(t, carry, acc=acc):
                for j in range(ncol):
                    v = acc[t, pl.ds(j * 16, 16)] + bias_v[pl.ds(j * 16, 16)]
                    if relu:
                        v = jnp.maximum(v, 0.0)
                    acc[t, pl.ds(j * 16, 16)] = v
                return carry

            lax.fori_loop(0, _CH, fin, 0)
            wh[c % 2] = pltpu.async_copy(
                acc, out_hbm.at[bb, pl.ds(base + c * _CH, _CH)], wsems[c % 2])
        for h in wh:
            if h is not None:
                h.wait()

    return gather


def _nn_layer(feat, loct, samplocs, Wk, b, relu):
    B, L, CF = feat.shape
    Cout = Wk.shape[0]
    Coutp = ((Cout + 15) // 16) * 16
    sampf = feat[:, _SIDX, :]
    idx = pl.pallas_call(
        _select_kernel,
        grid=(B, NBLK),
        in_specs=[
            pl.BlockSpec((1, TL, CF), lambda b_, i: (b_, i, 0)),
            pl.BlockSpec((TL, 8), lambda b_, i: (i, 0)),
            pl.BlockSpec((1, NN, CF), lambda b_, i: (b_, 0, 0)),
            pl.BlockSpec((NN, 8), lambda b_, i: (0, 0)),
        ],
        out_specs=pl.BlockSpec((KK, 1, TL, 1), lambda b_, i: (0, b_, i, 0)),
        out_shape=jax.ShapeDtypeStruct((KK, B, L, 1), jnp.int32),
    )(feat, loct, sampf, samplocs)
    idx = idx.reshape(KK * B * L)
    Wt = jnp.transpose(Wk, (2, 1, 0))
    Wtp = jnp.pad(Wt, ((0, 0), (0, 0), (0, Coutp - Cout)))
    tab = pl.pallas_call(
        _tab_kernel,
        grid=(B,),
        in_specs=[
            pl.BlockSpec((1, NN, CF), lambda b_: (b_, 0, 0)),
            pl.BlockSpec((NN, 8), lambda b_: (0, 0)),
            pl.BlockSpec((KK, CF, Coutp), lambda b_: (0, 0, 0)),
            pl.BlockSpec((KK, 8, Coutp), lambda b_: (0, 0, 0)),
        ],
        out_specs=pl.BlockSpec((1, KK, NN, Coutp), lambda b_: (b_, 0, 0, 0)),
        out_shape=jax.ShapeDtypeStruct((B, KK, NN, Coutp), jnp.float32),
    )(sampf, samplocs, Wtp[:, :CF, :], Wtp[:, CF:, :])
    tabflat = tab.reshape(B * KK * NN, Coutp)
    biasp = jnp.pad(b, (0, Coutp - Cout))
    out = _make_gather(B, Coutp, relu)(tabflat, idx, biasp)
    return out[:, :, :Cout] if Coutp != Cout else out


def kernel(x, W1, b1, W2, b2, W3, b3):
    B = x.shape[0]
    H = x.shape[2]
    W = x.shape[3]
    ys = jnp.linspace(-1.0, 1.0, H)
    xs = jnp.linspace(-1.0, 1.0, W)
    gy, gx = jnp.meshgrid(ys, xs, indexing='ij')
    loc = jnp.stack([gy, gx])[None]
    locu = _unshuffle(loc, SC)
    L = (H // SC) * (W // SC)
    loct = locu.reshape(2 * SC * SC, L).transpose(1, 0)
    samplocs = loct[jnp.array(_SIDX), :]

    xt = _unshuffle(x, SC)
    feat = xt.reshape(B, -1, L).transpose(0, 2, 1)

    o1 = _nn_layer(feat, loct, samplocs, W1, b1, relu=True)
    o2 = _nn_layer(o1, loct, samplocs, W2, b2, relu=True)
    o3 = _nn_layer(o2, loct, samplocs, W3, b3, relu=False)

    Cout = o3.shape[-1]
    out = o3.transpose(0, 2, 1).reshape(B, Cout, H // SC, W // SC)
    return _shuffle(out, SC)
